# Initial kernel scaffold; baseline (speedup 1.0000x reference)
import jax
import jax.numpy as jnp
from jax.experimental import pallas as pl


def kernel(x, edge_index, fc1_w, fc1_b, c1_w, c1_b, c2_w, c2_b, c31_w, c31_b, c32_w, c32_b, fc21_w, fc21_b, fc22_w, fc22_b):
    raise NotImplementedError("write your pallas kernel here")


# speedup vs baseline: 26.4367x; 26.4367x over previous
import functools

import jax
import jax.numpy as jnp
from jax import lax
from jax.experimental import pallas as pl
from jax.experimental.pallas import tpu as pltpu
from jax.experimental.pallas import tpu_sc as plsc

N = 100000
F = 32
HALF = 16
NT = 16
NSC = 2
CHUNK = 128
GRP = 8
N_PAD = 100096
BLK = 2000

_mesh = plsc.VectorSubcoreMesh(core_axis_name="c", subcore_axis_name="s")


def _agg_body(tbl, row2, col2, out, accum, rbuf, cbuf, dbuf, gsem, ssem):
    c = lax.axis_index("c")
    s = lax.axis_index("s")
    rpt = N_PAD // NT
    rbase = s * rpt
    tblc = tbl.at[c]
    pltpu.sync_copy(tblc.at[pl.ds(rbase, rpt)], accum.at[pl.ds(rbase, rpt)])
    plsc.subcore_barrier()

    nchunks = row2.shape[0]
    cpt = nchunks // NT
    groups = cpt // GRP
    gbase = s * cpt

    def group(g, carry):
        ch0 = gbase + g * GRP
        pltpu.sync_copy(row2.at[pl.ds(ch0, GRP)], rbuf)
        pltpu.sync_copy(col2.at[pl.ds(ch0, GRP)], cbuf)
        gds = [pltpu.async_copy(tblc.at[rbuf.at[j]], dbuf.at[j], gsem)
               for j in range(GRP)]
        for d in gds:
            d.wait()
        sds = [pltpu.async_copy(dbuf.at[j], accum.at[cbuf.at[j]], ssem,
                                add=True)
               for j in range(GRP)]
        for d in sds:
            d.wait()
        return carry

    lax.fori_loop(0, groups, group, 0)
    plsc.subcore_barrier()
    pltpu.sync_copy(accum.at[pl.ds(rbase, rpt)],
                    out.at[c].at[pl.ds(rbase, rpt)])


def _deg_body(col2, zeros, ones, degp, accum, cbuf, ones_v, ssem):
    c = lax.axis_index("c")
    s = lax.axis_index("s")
    rpt = N_PAD // NT
    rbase = s * rpt
    pltpu.sync_copy(zeros.at[pl.ds(rbase, rpt)], accum.at[pl.ds(rbase, rpt)])
    pltpu.sync_copy(ones, ones_v)
    plsc.subcore_barrier()

    w = c * NT + s
    nchunks = col2.shape[0]
    cpt = nchunks // (NSC * NT)
    groups = cpt // GRP
    gbase = w * cpt

    def group(g, carry):
        ch0 = gbase + g * GRP
        pltpu.sync_copy(col2.at[pl.ds(ch0, GRP)], cbuf)
        sds = [pltpu.async_copy(ones_v, accum.at[cbuf.at[j]], ssem, add=True)
               for j in range(GRP)]
        for d in sds:
            d.wait()
        return carry

    lax.fori_loop(0, groups, group, 0)
    plsc.subcore_barrier()
    pltpu.sync_copy(accum.at[pl.ds(rbase, rpt)],
                    degp.at[c].at[pl.ds(rbase, rpt)])


_sc_params = pltpu.CompilerParams(use_tc_tiling_on_sc=False)


def _make_agg(nchunks):
    return functools.partial(
        pl.kernel,
        out_type=jax.ShapeDtypeStruct((NSC, N_PAD, HALF), jnp.float32),
        mesh=_mesh,
        compiler_params=_sc_params,
        scratch_types=[
            pltpu.VMEM_SHARED((N_PAD, HALF), jnp.float32),
            pltpu.VMEM((GRP, CHUNK), jnp.int32),
            pltpu.VMEM((GRP, CHUNK), jnp.int32),
            pltpu.VMEM((GRP, CHUNK, HALF), jnp.float32),
            pltpu.SemaphoreType.DMA,
            pltpu.SemaphoreType.DMA,
        ],
    )(_agg_body)


def _make_deg():
    return functools.partial(
        pl.kernel,
        out_type=jax.ShapeDtypeStruct((NSC, N_PAD, 1), jnp.float32),
        mesh=_mesh,
        compiler_params=_sc_params,
        scratch_types=[
            pltpu.VMEM_SHARED((N_PAD, 1), jnp.float32),
            pltpu.VMEM((GRP, CHUNK), jnp.int32),
            pltpu.VMEM((CHUNK, 1), jnp.float32),
            pltpu.SemaphoreType.DMA,
        ],
    )(_deg_body)


def _pre_body(x_ref, w_ref, b_ref, degp_ref, hs_ref, dinv_ref):
    d = degp_ref[...]
    deg = d[0] + d[1] + 1.0
    dinv = lax.rsqrt(deg)
    h = lax.dot_general(x_ref[...], w_ref[...], (((1,), (1,)), ((), ())),
                        preferred_element_type=jnp.float32)
    h = jnp.maximum(h + b_ref[...], 0.0)
    hs = dinv * h
    dinv_ref[...] = dinv
    hs_ref[0] = hs[:, :HALF]
    hs_ref[1] = hs[:, HALF:]


def _mid_body(m2_ref, dinv_ref, w_ref, b_ref, hs_ref):
    m = jnp.concatenate([m2_ref[0], m2_ref[1]], axis=1)
    dinv = dinv_ref[...]
    a = dinv * m
    h = lax.dot_general(a, w_ref[...], (((1,), (1,)), ((), ())),
                        preferred_element_type=jnp.float32)
    h = jnp.maximum(h + b_ref[...], 0.0)
    hs = dinv * h
    hs_ref[0] = hs[:, :HALF]
    hs_ref[1] = hs[:, HALF:]


def _fin_body(m2_ref, dinv_ref, w31_ref, b31_ref, w32_ref, b32_ref,
              w2_ref, b2_ref, out_ref):
    m = jnp.concatenate([m2_ref[0], m2_ref[1]], axis=1)
    dinv = dinv_ref[...]
    a = dinv * m
    dn = (((1,), (1,)), ((), ()))
    h1 = jnp.maximum(
        lax.dot_general(a, w31_ref[...], dn,
                        preferred_element_type=jnp.float32) + b31_ref[...],
        0.0)
    h2 = jnp.maximum(
        lax.dot_general(a, w32_ref[...], dn,
                        preferred_element_type=jnp.float32) + b32_ref[...],
        0.0)
    h_cat = jnp.concatenate([h1, h2], axis=1)
    out_ref[...] = lax.dot_general(
        h_cat, w2_ref[...], dn,
        preferred_element_type=jnp.float32) + b2_ref[...]


def _full(shape):
    return pl.BlockSpec(shape, lambda i: (0,) * len(shape))


_row_spec2 = pl.BlockSpec((BLK, 2), lambda i: (i, 0))
_hs_spec = pl.BlockSpec((NSC, BLK, HALF), lambda i: (0, i, 0))
_dinv_spec = pl.BlockSpec((BLK, 1), lambda i: (i, 0))
_degp_spec = pl.BlockSpec((NSC, BLK, 1), lambda i: (0, i, 0))


def _pre_call(x, w, b, degp):
    return pl.pallas_call(
        _pre_body,
        grid=(N // BLK,),
        in_specs=[_row_spec2, _full((F, 2)), _full((1, F)), _degp_spec],
        out_specs=[_hs_spec, _dinv_spec],
        out_shape=[
            jax.ShapeDtypeStruct((NSC, N_PAD, HALF), jnp.float32),
            jax.ShapeDtypeStruct((N_PAD, 1), jnp.float32),
        ],
    )(x, w, b, degp)


def _mid_call(m2, dinv, w, b):
    return pl.pallas_call(
        _mid_body,
        grid=(N // BLK,),
        in_specs=[_hs_spec, _dinv_spec, _full((F, F)), _full((1, F))],
        out_specs=_hs_spec,
        out_shape=jax.ShapeDtypeStruct((NSC, N_PAD, HALF), jnp.float32),
    )(m2, dinv, w, b)


def _fin_call(m2, dinv, w31, b31, w32, b32, w2, b2):
    return pl.pallas_call(
        _fin_body,
        grid=(N // BLK,),
        in_specs=[_hs_spec, _dinv_spec,
                  _full((F, F)), _full((1, F)), _full((F, F)), _full((1, F)),
                  _full((2, 2 * F)), _full((1, 2))],
        out_specs=pl.BlockSpec((BLK, 2), lambda i: (i, 0)),
        out_shape=jax.ShapeDtypeStruct((N, 2), jnp.float32),
    )(m2, dinv, w31, b31, w32, b32, w2, b2)


def kernel(x, edge_index, fc1_w, fc1_b, c1_w, c1_b, c2_w, c2_b,
           c31_w, c31_b, c32_w, c32_b, fc21_w, fc21_b, fc22_w, fc22_b):
    e = edge_index.shape[1]
    align = NSC * NT * CHUNK * GRP
    e_pad = ((e + align - 1) // align) * align
    npad = e_pad - e
    ei = edge_index.astype(jnp.int32)
    pad_idx = (jnp.arange(npad, dtype=jnp.int32) % (N_PAD - N)) + N
    row2 = jnp.concatenate([ei[0], pad_idx]).reshape(e_pad // CHUNK, CHUNK)
    col2 = jnp.concatenate([ei[1], pad_idx]).reshape(e_pad // CHUNK, CHUNK)

    zeros = jnp.zeros((N_PAD, 1), jnp.float32)
    ones = jnp.ones((CHUNK, 1), jnp.float32)
    degp = _make_deg()(col2, zeros, ones)

    hs, dinv = _pre_call(x, fc1_w, fc1_b.reshape(1, F), degp)
    agg = _make_agg(e_pad // CHUNK)
    m = agg(hs, row2, col2)
    hs = _mid_call(m, dinv, c1_w, c1_b.reshape(1, F))
    m = agg(hs, row2, col2)
    hs = _mid_call(m, dinv, c2_w, c2_b.reshape(1, F))
    m = agg(hs, row2, col2)
    w2 = jnp.zeros((2, 2 * F), jnp.float32)
    w2 = w2.at[0, :F].set(fc21_w[0]).at[1, F:].set(fc22_w[0])
    b2 = jnp.concatenate([fc21_b, fc22_b]).reshape(1, 2)
    return _fin_call(m, dinv, c31_w, c31_b.reshape(1, F),
                     c32_w, c32_b.reshape(1, F), w2, b2)


# speedup vs baseline: 40.3781x; 1.5273x over previous
import functools

import jax
import jax.numpy as jnp
import numpy as np
from jax import lax
from jax.experimental import pallas as pl
from jax.experimental.pallas import tpu as pltpu
from jax.experimental.pallas import tpu_sc as plsc

N = 100000
F = 32
HALF = 16
NT = 16
NSC = 2
CHUNK = 128
GRP = 8
N_PAD = 100096
NPK = N_PAD // 8
PBLK = NPK // 4

_mesh = plsc.VectorSubcoreMesh(core_axis_name="c", subcore_axis_name="s")
_sc_params = pltpu.CompilerParams(use_tc_tiling_on_sc=False)


def _agg_body(tbl, row2, col2, out, accum, rbuf, cbuf, dbuf, gsem, ssem):
    c = lax.axis_index("c")
    s = lax.axis_index("s")
    rpt = N_PAD // NT
    rbase = s * rpt
    tblc = tbl.at[c]
    pltpu.sync_copy(tblc.at[pl.ds(rbase, rpt)], accum.at[pl.ds(rbase, rpt)])
    plsc.subcore_barrier()

    nchunks = row2.shape[0]
    cpt = nchunks // NT
    groups = cpt // GRP
    gbase = s * cpt

    def group(g, carry):
        ch0 = gbase + g * GRP
        pltpu.sync_copy(row2.at[pl.ds(ch0, GRP)], rbuf)
        pltpu.sync_copy(col2.at[pl.ds(ch0, GRP)], cbuf)
        gds = [pltpu.async_copy(tblc.at[rbuf.at[j]], dbuf.at[j], gsem)
               for j in range(GRP)]
        for d in gds:
            d.wait()
        sds = [pltpu.async_copy(dbuf.at[j], accum.at[cbuf.at[j]], ssem,
                                add=True)
               for j in range(GRP)]
        for d in sds:
            d.wait()
        return carry

    lax.fori_loop(0, groups, group, 0)
    plsc.subcore_barrier()
    pltpu.sync_copy(accum.at[pl.ds(rbase, rpt)],
                    out.at[c].at[pl.ds(rbase, rpt)])


def _deg_body(col2, zeros, ones, degp, accum, cbuf, ones_v, ssem):
    c = lax.axis_index("c")
    s = lax.axis_index("s")
    rpt = N_PAD // NT
    rbase = s * rpt
    pltpu.sync_copy(zeros.at[pl.ds(rbase, rpt)], accum.at[pl.ds(rbase, rpt)])
    pltpu.sync_copy(ones, ones_v)
    plsc.subcore_barrier()

    w = c * NT + s
    nchunks = col2.shape[0]
    cpt = nchunks // (NSC * NT)
    groups = cpt // GRP
    gbase = w * cpt

    def group(g, carry):
        ch0 = gbase + g * GRP
        pltpu.sync_copy(col2.at[pl.ds(ch0, GRP)], cbuf)
        sds = [pltpu.async_copy(ones_v, accum.at[cbuf.at[j]], ssem, add=True)
               for j in range(GRP)]
        for d in sds:
            d.wait()
        return carry

    lax.fori_loop(0, groups, group, 0)
    plsc.subcore_barrier()
    pltpu.sync_copy(accum.at[pl.ds(rbase, rpt)],
                    degp.at[c].at[pl.ds(rbase, rpt)])


_agg_call = functools.partial(
    pl.kernel,
    out_type=jax.ShapeDtypeStruct((NSC, N_PAD, HALF), jnp.float32),
    mesh=_mesh,
    compiler_params=_sc_params,
    scratch_types=[
        pltpu.VMEM_SHARED((N_PAD, HALF), jnp.float32),
        pltpu.VMEM((GRP, CHUNK), jnp.int32),
        pltpu.VMEM((GRP, CHUNK), jnp.int32),
        pltpu.VMEM((GRP, CHUNK, HALF), jnp.float32),
        pltpu.SemaphoreType.DMA,
        pltpu.SemaphoreType.DMA,
    ],
)(_agg_body)


_deg_call = functools.partial(
    pl.kernel,
    out_type=jax.ShapeDtypeStruct((NSC, N_PAD, HALF), jnp.float32),
    mesh=_mesh,
    compiler_params=_sc_params,
    scratch_types=[
        pltpu.VMEM_SHARED((N_PAD, HALF), jnp.float32),
        pltpu.VMEM((GRP, CHUNK), jnp.int32),
        pltpu.VMEM((CHUNK, HALF), jnp.float32),
        pltpu.SemaphoreType.DMA,
    ],
)(_deg_body)


def _pre_body(x0_ref, x1_ref, dp_ref, w0_ref, w1_ref, bb_ref,
              hs_ref, dinv_ref):
    d = dp_ref[...]
    deg = d[0] + d[1] + 1.0
    dinvp = lax.rsqrt(deg)
    dinv_ref[...] = dinvp
    x0 = x0_ref[...]
    x1 = x1_ref[...]
    for c in range(NSC):
        h = jnp.maximum(x0 * w0_ref[c] + x1 * w1_ref[c] + bb_ref[c], 0.0)
        hs_ref[c] = dinvp * h


def _mid_body(m_ref, dinv_ref, b4_ref, bb_ref, hs_ref):
    dinvp = dinv_ref[...]
    dn = (((1,), (0,)), ((), ()))
    a0 = dinvp * m_ref[0]
    a1 = dinvp * m_ref[1]
    for c in range(NSC):
        q = (lax.dot_general(a0, b4_ref[c, 0], dn,
                             preferred_element_type=jnp.float32)
             + lax.dot_general(a1, b4_ref[c, 1], dn,
                               preferred_element_type=jnp.float32))
        h = jnp.maximum(q + bb_ref[c], 0.0)
        hs_ref[c] = dinvp * h


def _fin_body(m_ref, dinv_ref, b31_ref, bb31_ref, b32_ref, bb32_ref,
              wh_ref, sel_ref, bout_ref, out_ref):
    dinvp = dinv_ref[...]
    dn = (((1,), (0,)), ((), ()))
    a0 = dinvp * m_ref[0]
    a1 = dinvp * m_ref[1]
    acc = None
    for (b_ref, bb_ref, head) in ((b31_ref, bb31_ref, 0),
                                  (b32_ref, bb32_ref, 1)):
        for c in range(NSC):
            q = (lax.dot_general(a0, b_ref[c, 0], dn,
                                 preferred_element_type=jnp.float32)
                 + lax.dot_general(a1, b_ref[c, 1], dn,
                                   preferred_element_type=jnp.float32))
            h = jnp.maximum(q + bb_ref[c], 0.0)
            t = h * wh_ref[head, c]
            part = lax.dot_general(t, sel_ref[head], dn,
                                   preferred_element_type=jnp.float32)
            acc = part if acc is None else acc + part
    out_ref[...] = acc + bout_ref[...]


def _pk_spec():
    return pl.BlockSpec((PBLK, 128), lambda i: (i, 0))


def _pk2_spec():
    return pl.BlockSpec((NSC, PBLK, 128), lambda i: (0, i, 0))


def _full(shape):
    return pl.BlockSpec(shape, lambda i: (0,) * len(shape))


def _pre_call(x0, x1, dp, w0, w1, bb):
    return pl.pallas_call(
        _pre_body,
        grid=(NPK // PBLK,),
        in_specs=[_pk_spec(), _pk_spec(), _pk2_spec(),
                  _full((NSC, 1, 128)), _full((NSC, 1, 128)),
                  _full((NSC, 1, 128))],
        out_specs=[_pk2_spec(), _pk_spec()],
        out_shape=[
            jax.ShapeDtypeStruct((NSC, NPK, 128), jnp.float32),
            jax.ShapeDtypeStruct((NPK, 128), jnp.float32),
        ],
    )(x0, x1, dp, w0, w1, bb)


def _mid_call(mp, dinvp, b4, bb):
    return pl.pallas_call(
        _mid_body,
        grid=(NPK // PBLK,),
        in_specs=[_pk2_spec(), _pk_spec(),
                  _full((NSC, NSC, 128, 128)), _full((NSC, 1, 128))],
        out_specs=_pk2_spec(),
        out_shape=jax.ShapeDtypeStruct((NSC, NPK, 128), jnp.float32),
    )(mp, dinvp, b4, bb)


def _fin_call(mp, dinvp, b31, bb31, b32, bb32, wh, sel, bout):
    return pl.pallas_call(
        _fin_body,
        grid=(NPK // PBLK,),
        in_specs=[_pk2_spec(), _pk_spec(),
                  _full((NSC, NSC, 128, 128)), _full((NSC, 1, 128)),
                  _full((NSC, NSC, 128, 128)), _full((NSC, 1, 128)),
                  _full((2, NSC, 1, 128)), _full((2, 128, 16)),
                  _full((1, 16))],
        out_specs=pl.BlockSpec((PBLK, 16), lambda i: (i, 0)),
        out_shape=jax.ShapeDtypeStruct((NPK, 16), jnp.float32),
    )(mp, dinvp, b31, bb31, b32, bb32, wh, sel, bout)


def _blockdiag(w):
    eye = jnp.eye(8, dtype=jnp.float32)
    blocks = []
    for co in range(NSC):
        rows = []
        for ci in range(NSC):
            blk = w[co * HALF:(co + 1) * HALF, ci * HALF:(ci + 1) * HALF]
            rows.append(jnp.kron(eye, blk.T))
        blocks.append(jnp.stack(rows))
    return jnp.stack(blocks)


def _brc(v):
    t = v.reshape(NSC, 1, HALF)
    return jnp.tile(t, (1, 1, 8)).reshape(NSC, 1, 128)


_SEL = np.zeros((2, 128, 16), np.float32)
for _g in range(8):
    for _l in range(16):
        _SEL[0, _g * 16 + _l, 2 * _g] = 1.0
        _SEL[1, _g * 16 + _l, 2 * _g + 1] = 1.0


def _pack_nodes(v, fill):
    vp = jnp.full((N_PAD,), fill, v.dtype).at[:N].set(v)
    return jnp.repeat(vp, HALF).reshape(NPK, 128)


def kernel(x, edge_index, fc1_w, fc1_b, c1_w, c1_b, c2_w, c2_b,
           c31_w, c31_b, c32_w, c32_b, fc21_w, fc21_b, fc22_w, fc22_b):
    e = edge_index.shape[1]
    align = NSC * NT * CHUNK * GRP
    e_pad = ((e + align - 1) // align) * align
    npad = e_pad - e
    ei = edge_index.astype(jnp.int32)
    pad_idx = (jnp.arange(npad, dtype=jnp.int32) % (N_PAD - N)) + N
    row2 = jnp.concatenate([ei[0], pad_idx]).reshape(e_pad // CHUNK, CHUNK)
    col2 = jnp.concatenate([ei[1], pad_idx]).reshape(e_pad // CHUNK, CHUNK)

    zeros = jnp.zeros((N_PAD, HALF), jnp.float32)
    ones = jnp.ones((CHUNK, HALF), jnp.float32)
    degp = _deg_call(col2, zeros, ones)
    dp = degp.reshape(NSC, NPK, 128)

    x0 = _pack_nodes(x[:, 0], 0.0)
    x1 = _pack_nodes(x[:, 1], 0.0)
    w0 = _brc(fc1_w[:, 0])
    w1 = _brc(fc1_w[:, 1])

    hsp, dinvp = _pre_call(x0, x1, dp, w0, w1, _brc(fc1_b))
    m = _agg_call(hsp.reshape(NSC, N_PAD, HALF), row2, col2)
    hsp = _mid_call(m.reshape(NSC, NPK, 128), dinvp, _blockdiag(c1_w),
                    _brc(c1_b))
    m = _agg_call(hsp.reshape(NSC, N_PAD, HALF), row2, col2)
    hsp = _mid_call(m.reshape(NSC, NPK, 128), dinvp, _blockdiag(c2_w),
                    _brc(c2_b))
    m = _agg_call(hsp.reshape(NSC, N_PAD, HALF), row2, col2)

    wh = jnp.stack([_brc(fc21_w[0]), _brc(fc22_w[0])])
    sel = jnp.asarray(_SEL)
    bout = jnp.tile(
        jnp.concatenate([fc21_b, fc22_b]).reshape(1, 2), (1, 8))
    outp = _fin_call(m.reshape(NSC, NPK, 128), dinvp, _blockdiag(c31_w),
                     _brc(c31_b), _blockdiag(c32_w), _brc(c32_b),
                     wh, sel, bout)
    return outp.reshape(N_PAD, 2)[:N]


# speedup vs baseline: 51.3678x; 1.2722x over previous
import functools

import jax
import jax.numpy as jnp
import numpy as np
from jax import lax
from jax.experimental import pallas as pl
from jax.experimental.pallas import tpu as pltpu
from jax.experimental.pallas import tpu_sc as plsc

N = 100000
F = 32
HALF = 16
NT = 16
NSC = 2
CHUNK = 128
GRP = 6
N_PAD = 100096
NPK = N_PAD // 8
PBLK = NPK // 4

_mesh = plsc.VectorSubcoreMesh(core_axis_name="c", subcore_axis_name="s")
_sc_params = pltpu.CompilerParams(use_tc_tiling_on_sc=False)


def _agg_body(tbl, row1, col2, out, accum,
              rbuf0, cbuf0, dbuf0, rbuf1, cbuf1, dbuf1,
              gsem0, ssem0, gsem1, ssem1):
    c = lax.axis_index("c")
    s = lax.axis_index("s")
    rpt = N_PAD // NT
    rbase = s * rpt
    tblc = tbl.at[c]
    pltpu.sync_copy(tblc.at[pl.ds(rbase, rpt)], accum.at[pl.ds(rbase, rpt)])
    plsc.subcore_barrier()

    nchunks = col2.shape[0]
    cpt = nchunks // NT
    groups = cpt // GRP
    gbase = s * cpt
    slabs = ((rbuf0, cbuf0, dbuf0, gsem0, ssem0),
             (rbuf1, cbuf1, dbuf1, gsem1, ssem1))

    def load_and_gather(g, slab):
        rbuf, cbuf, dbuf, gsem, _ = slab
        ch0 = gbase + g * GRP
        pltpu.sync_copy(row1.at[pl.ds(ch0 * CHUNK, GRP * CHUNK)], rbuf)
        pltpu.sync_copy(col2.at[pl.ds(ch0, GRP)], cbuf)
        pltpu.async_copy(tblc.at[rbuf], dbuf, gsem)

    def fire_scatters(slab):
        rbuf, cbuf, dbuf, gsem, ssem = slab
        pltpu.make_async_copy(tblc.at[rbuf], dbuf, gsem).wait()
        for j in range(GRP):
            pltpu.async_copy(dbuf.at[pl.ds(j * CHUNK, CHUNK)],
                             accum.at[cbuf.at[j]], ssem, add=True)

    def wait_scatters(slab):
        rbuf, cbuf, dbuf, gsem, ssem = slab
        for j in range(GRP):
            pltpu.make_async_copy(dbuf.at[pl.ds(j * CHUNK, CHUNK)],
                                  accum.at[cbuf.at[j]], ssem).wait()

    load_and_gather(0, slabs[0])
    load_and_gather(1, slabs[1])

    def body(it, carry):
        g0 = 2 * it
        fire_scatters(slabs[0])
        fire_scatters(slabs[1])
        wait_scatters(slabs[0])

        @pl.when(g0 + 2 < groups)
        def _():
            load_and_gather(g0 + 2, slabs[0])

        wait_scatters(slabs[1])

        @pl.when(g0 + 3 < groups)
        def _():
            load_and_gather(g0 + 3, slabs[1])

        return carry

    lax.fori_loop(0, groups // 2, body, 0)
    plsc.subcore_barrier()
    pltpu.sync_copy(accum.at[pl.ds(rbase, rpt)],
                    out.at[c].at[pl.ds(rbase, rpt)])


def _deg_body(col2, zeros, ones, degp, accum, cbuf0, cbuf1, ones_v,
              ssem0, ssem1):
    c = lax.axis_index("c")
    s = lax.axis_index("s")
    rpt = N_PAD // NT
    rbase = s * rpt
    pltpu.sync_copy(zeros.at[pl.ds(rbase, rpt)], accum.at[pl.ds(rbase, rpt)])
    pltpu.sync_copy(ones, ones_v)
    plsc.subcore_barrier()

    w = c * NT + s
    nchunks = col2.shape[0]
    cpt = nchunks // (NSC * NT)
    groups = cpt // GRP
    gbase = w * cpt
    slabs = ((cbuf0, ssem0), (cbuf1, ssem1))

    def fire(g, slab):
        cbuf, ssem = slab
        pltpu.sync_copy(col2.at[pl.ds(gbase + g * GRP, GRP)], cbuf)
        for j in range(GRP):
            pltpu.async_copy(ones_v, accum.at[cbuf.at[j]], ssem, add=True)

    def drain(slab):
        cbuf, ssem = slab
        for j in range(GRP):
            pltpu.make_async_copy(ones_v, accum.at[cbuf.at[j]], ssem).wait()

    fire(0, slabs[0])

    def body(it, carry):
        g0 = 2 * it
        fire(g0 + 1, slabs[1])
        drain(slabs[0])

        @pl.when(g0 + 2 < groups)
        def _():
            fire(g0 + 2, slabs[0])

        drain(slabs[1])
        return carry

    lax.fori_loop(0, groups // 2, body, 0)
    plsc.subcore_barrier()
    pltpu.sync_copy(accum.at[pl.ds(rbase, rpt)],
                    degp.at[c].at[pl.ds(rbase, rpt)])


_agg_call = functools.partial(
    pl.kernel,
    out_type=jax.ShapeDtypeStruct((NSC, N_PAD, HALF), jnp.float32),
    mesh=_mesh,
    compiler_params=_sc_params,
    scratch_types=[
        pltpu.VMEM_SHARED((N_PAD, HALF), jnp.float32),
        pltpu.VMEM((GRP * CHUNK,), jnp.int32),
        pltpu.VMEM((GRP, CHUNK), jnp.int32),
        pltpu.VMEM((GRP * CHUNK, HALF), jnp.float32),
        pltpu.VMEM((GRP * CHUNK,), jnp.int32),
        pltpu.VMEM((GRP, CHUNK), jnp.int32),
        pltpu.VMEM((GRP * CHUNK, HALF), jnp.float32),
        pltpu.SemaphoreType.DMA,
        pltpu.SemaphoreType.DMA,
        pltpu.SemaphoreType.DMA,
        pltpu.SemaphoreType.DMA,
    ],
)(_agg_body)


_deg_call = functools.partial(
    pl.kernel,
    out_type=jax.ShapeDtypeStruct((NSC, N_PAD, HALF), jnp.float32),
    mesh=_mesh,
    compiler_params=_sc_params,
    scratch_types=[
        pltpu.VMEM_SHARED((N_PAD, HALF), jnp.float32),
        pltpu.VMEM((GRP, CHUNK), jnp.int32),
        pltpu.VMEM((GRP, CHUNK), jnp.int32),
        pltpu.VMEM((CHUNK, HALF), jnp.float32),
        pltpu.SemaphoreType.DMA,
        pltpu.SemaphoreType.DMA,
    ],
)(_deg_body)


def _pre_body(x0_ref, x1_ref, dp_ref, w0_ref, w1_ref, bb_ref,
              hs_ref, dinv_ref):
    d = dp_ref[...]
    deg = d[0] + d[1] + 1.0
    dinvp = lax.rsqrt(deg)
    dinv_ref[...] = dinvp
    x0 = x0_ref[...]
    x1 = x1_ref[...]
    for c in range(NSC):
        h = jnp.maximum(x0 * w0_ref[c] + x1 * w1_ref[c] + bb_ref[c], 0.0)
        hs_ref[c] = dinvp * h


def _mid_body(m_ref, dinv_ref, b4_ref, bb_ref, hs_ref):
    dinvp = dinv_ref[...]
    dn = (((1,), (0,)), ((), ()))
    a0 = dinvp * m_ref[0]
    a1 = dinvp * m_ref[1]
    for c in range(NSC):
        q = (lax.dot_general(a0, b4_ref[c, 0], dn,
                             preferred_element_type=jnp.float32)
             + lax.dot_general(a1, b4_ref[c, 1], dn,
                               preferred_element_type=jnp.float32))
        h = jnp.maximum(q + bb_ref[c], 0.0)
        hs_ref[c] = dinvp * h


def _fin_body(m_ref, dinv_ref, b31_ref, bb31_ref, b32_ref, bb32_ref,
              wh_ref, sel_ref, bout_ref, out_ref):
    dinvp = dinv_ref[...]
    dn = (((1,), (0,)), ((), ()))
    a0 = dinvp * m_ref[0]
    a1 = dinvp * m_ref[1]
    acc = None
    for (b_ref, bb_ref, head) in ((b31_ref, bb31_ref, 0),
                                  (b32_ref, bb32_ref, 1)):
        for c in range(NSC):
            q = (lax.dot_general(a0, b_ref[c, 0], dn, preferred_element_type=jnp.float32)
                 + lax.dot_general(a1, b_ref[c, 1], dn,
                                   preferred_element_type=jnp.float32))
            h = jnp.maximum(q + bb_ref[c], 0.0)
            t = h * wh_ref[head, c]
            part = lax.dot_general(t, sel_ref[head], dn,
                                   preferred_element_type=jnp.float32)
            acc = part if acc is None else acc + part
    out_ref[...] = acc + bout_ref[...]


def _pk_spec():
    return pl.BlockSpec((PBLK, 128), lambda i: (i, 0))


def _pk2_spec():
    return pl.BlockSpec((NSC, PBLK, 128), lambda i: (0, i, 0))


def _full(shape):
    return pl.BlockSpec(shape, lambda i: (0,) * len(shape))


def _pre_call(x0, x1, dp, w0, w1, bb):
    return pl.pallas_call(
        _pre_body,
        grid=(NPK // PBLK,),
        in_specs=[_pk_spec(), _pk_spec(), _pk2_spec(),
                  _full((NSC, 1, 128)), _full((NSC, 1, 128)),
                  _full((NSC, 1, 128))],
        out_specs=[_pk2_spec(), _pk_spec()],
        out_shape=[
            jax.ShapeDtypeStruct((NSC, NPK, 128), jnp.float32),
            jax.ShapeDtypeStruct((NPK, 128), jnp.float32),
        ],
    )(x0, x1, dp, w0, w1, bb)


def _mid_call(mp, dinvp, b4, bb):
    return pl.pallas_call(
        _mid_body,
        grid=(NPK // PBLK,),
        in_specs=[_pk2_spec(), _pk_spec(),
                  _full((NSC, NSC, 128, 128)), _full((NSC, 1, 128))],
        out_specs=_pk2_spec(),
        out_shape=jax.ShapeDtypeStruct((NSC, NPK, 128), jnp.float32),
    )(mp, dinvp, b4, bb)


def _fin_call(mp, dinvp, b31, bb31, b32, bb32, wh, sel, bout):
    return pl.pallas_call(
        _fin_body,
        grid=(NPK // PBLK,),
        in_specs=[_pk2_spec(), _pk_spec(),
                  _full((NSC, NSC, 128, 128)), _full((NSC, 1, 128)),
                  _full((NSC, NSC, 128, 128)), _full((NSC, 1, 128)),
                  _full((2, NSC, 1, 128)), _full((2, 128, 16)),
                  _full((1, 16))],
        out_specs=pl.BlockSpec((PBLK, 16), lambda i: (i, 0)),
        out_shape=jax.ShapeDtypeStruct((NPK, 16), jnp.float32),
    )(mp, dinvp, b31, bb31, b32, bb32, wh, sel, bout)


def _blockdiag(w):
    eye = jnp.eye(8, dtype=jnp.float32)
    blocks = []
    for co in range(NSC):
        rows = []
        for ci in range(NSC):
            blk = w[co * HALF:(co + 1) * HALF, ci * HALF:(ci + 1) * HALF]
            rows.append(jnp.kron(eye, blk.T))
        blocks.append(jnp.stack(rows))
    return jnp.stack(blocks)


def _brc(v):
    t = v.reshape(NSC, 1, HALF)
    return jnp.tile(t, (1, 1, 8)).reshape(NSC, 1, 128)


_SEL = np.zeros((2, 128, 16), np.float32)
for _g in range(8):
    for _l in range(16):
        _SEL[0, _g * 16 + _l, 2 * _g] = 1.0
        _SEL[1, _g * 16 + _l, 2 * _g + 1] = 1.0


def _pack_nodes(v, fill):
    vp = jnp.full((N_PAD,), fill, v.dtype).at[:N].set(v)
    return jnp.repeat(vp, HALF).reshape(NPK, 128)


def kernel(x, edge_index, fc1_w, fc1_b, c1_w, c1_b, c2_w, c2_b,
           c31_w, c31_b, c32_w, c32_b, fc21_w, fc21_b, fc22_w, fc22_b):
    e = edge_index.shape[1]
    align = NSC * NT * CHUNK * GRP * 2
    e_pad = ((e + align - 1) // align) * align
    npad = e_pad - e
    ei = edge_index.astype(jnp.int32)
    pad_idx = (jnp.arange(npad, dtype=jnp.int32) % (N_PAD - N)) + N
    row1 = jnp.concatenate([ei[0], pad_idx])
    col2 = jnp.concatenate([ei[1], pad_idx]).reshape(e_pad // CHUNK, CHUNK)

    zeros = jnp.zeros((N_PAD, HALF), jnp.float32)
    ones = jnp.ones((CHUNK, HALF), jnp.float32)
    degp = _deg_call(col2, zeros, ones)
    dp = degp.reshape(NSC, NPK, 128)

    x0 = _pack_nodes(x[:, 0], 0.0)
    x1 = _pack_nodes(x[:, 1], 0.0)
    w0 = _brc(fc1_w[:, 0])
    w1 = _brc(fc1_w[:, 1])

    hsp, dinvp = _pre_call(x0, x1, dp, w0, w1, _brc(fc1_b))
    m = _agg_call(hsp.reshape(NSC, N_PAD, HALF), row1, col2)
    hsp = _mid_call(m.reshape(NSC, NPK, 128), dinvp, _blockdiag(c1_w),
                    _brc(c1_b))
    m = _agg_call(hsp.reshape(NSC, N_PAD, HALF), row1, col2)
    hsp = _mid_call(m.reshape(NSC, NPK, 128), dinvp, _blockdiag(c2_w),
                    _brc(c2_b))
    m = _agg_call(hsp.reshape(NSC, N_PAD, HALF), row1, col2)

    wh = jnp.stack([_brc(fc21_w[0]), _brc(fc22_w[0])])
    sel = jnp.asarray(_SEL)
    bout = jnp.tile(
        jnp.concatenate([fc21_b, fc22_b]).reshape(1, 2), (1, 8))
    outp = _fin_call(m.reshape(NSC, NPK, 128), dinvp, _blockdiag(c31_w),
                     _brc(c31_b), _blockdiag(c32_w), _brc(c32_b),
                     wh, sel, bout)
    return outp.reshape(N_PAD, 2)[:N]


# speedup vs baseline: 56.2323x; 1.0947x over previous
import functools

import jax
import jax.numpy as jnp
import numpy as np
from jax import lax
from jax.experimental import pallas as pl
from jax.experimental.pallas import tpu as pltpu
from jax.experimental.pallas import tpu_sc as plsc

N = 100000
F = 32
HALF = 16
NT = 16
NSC = 2
CHUNK = 128
GRP = 6
N_PAD = 100096
NPK = N_PAD // 8
PBLK = NPK // 4

_mesh = plsc.VectorSubcoreMesh(core_axis_name="c", subcore_axis_name="s")
_sc_params = pltpu.CompilerParams(use_tc_tiling_on_sc=False)


def _agg_body(tbl, row1, col2, out, accum,
              rbuf0, cbuf0, dbuf0, rbuf1, cbuf1, dbuf1,
              gsem0, ssem0, gsem1, ssem1):
    c = lax.axis_index("c")
    s = lax.axis_index("s")
    rpt = N_PAD // NT
    rbase = s * rpt
    tblc = tbl.at[c]
    pltpu.sync_copy(tblc.at[pl.ds(rbase, rpt)], accum.at[pl.ds(rbase, rpt)])
    plsc.subcore_barrier()

    nchunks = col2.shape[0]
    cpt = nchunks // NT
    groups = cpt // GRP
    gbase = s * cpt
    slabs = ((rbuf0, cbuf0, dbuf0, gsem0, ssem0),
             (rbuf1, cbuf1, dbuf1, gsem1, ssem1))

    def load_and_gather(g, slab):
        rbuf, cbuf, dbuf, gsem, _ = slab
        ch0 = gbase + g * GRP
        pltpu.sync_copy(row1.at[pl.ds(ch0 * CHUNK, GRP * CHUNK)], rbuf)
        pltpu.sync_copy(col2.at[pl.ds(ch0, GRP)], cbuf)
        pltpu.async_copy(tblc.at[rbuf], dbuf, gsem)

    def drain(g, slab):
        rbuf, cbuf, dbuf, gsem, ssem = slab
        pltpu.make_async_copy(tblc.at[rbuf], dbuf, gsem).wait()
        sds = [pltpu.async_copy(dbuf.at[pl.ds(j * CHUNK, CHUNK)],
                                accum.at[cbuf.at[j]], ssem, add=True)
               for j in range(GRP)]
        for d in sds:
            d.wait()

    load_and_gather(0, slabs[0])

    def body(it, carry):
        g0 = 2 * it
        load_and_gather(g0 + 1, slabs[1])
        drain(g0, slabs[0])

        @pl.when(g0 + 2 < groups)
        def _():
            load_and_gather(g0 + 2, slabs[0])

        drain(g0 + 1, slabs[1])
        return carry

    lax.fori_loop(0, groups // 2, body, 0)
    plsc.subcore_barrier()
    pltpu.sync_copy(accum.at[pl.ds(rbase, rpt)],
                    out.at[c].at[pl.ds(rbase, rpt)])


def _deg_body(col2, zeros, ones, degp, accum, cbuf0, cbuf1, ones_v,
              ssem0, ssem1):
    c = lax.axis_index("c")
    s = lax.axis_index("s")
    rpt = N_PAD // NT
    rbase = s * rpt
    pltpu.sync_copy(zeros.at[pl.ds(rbase, rpt)], accum.at[pl.ds(rbase, rpt)])
    pltpu.sync_copy(ones, ones_v)
    plsc.subcore_barrier()

    w = c * NT + s
    nchunks = col2.shape[0]
    cpt = nchunks // (NSC * NT)
    groups = cpt // GRP
    gbase = w * cpt
    slabs = ((cbuf0, ssem0), (cbuf1, ssem1))

    def fire(g, slab):
        cbuf, ssem = slab
        pltpu.sync_copy(col2.at[pl.ds(gbase + g * GRP, GRP)], cbuf)
        for j in range(GRP):
            pltpu.async_copy(ones_v, accum.at[cbuf.at[j]], ssem, add=True)

    def drain(slab):
        cbuf, ssem = slab
        for j in range(GRP):
            pltpu.make_async_copy(ones_v, accum.at[cbuf.at[j]], ssem).wait()

    fire(0, slabs[0])

    def body(it, carry):
        g0 = 2 * it
        fire(g0 + 1, slabs[1])
        drain(slabs[0])

        @pl.when(g0 + 2 < groups)
        def _():
            fire(g0 + 2, slabs[0])

        drain(slabs[1])
        return carry

    lax.fori_loop(0, groups // 2, body, 0)
    plsc.subcore_barrier()
    pltpu.sync_copy(accum.at[pl.ds(rbase, rpt)],
                    degp.at[c].at[pl.ds(rbase, rpt)])


_agg_call = functools.partial(
    pl.kernel,
    out_type=jax.ShapeDtypeStruct((NSC, N_PAD, HALF), jnp.float32),
    mesh=_mesh,
    compiler_params=_sc_params,
    scratch_types=[
        pltpu.VMEM_SHARED((N_PAD, HALF), jnp.float32),
        pltpu.VMEM((GRP * CHUNK,), jnp.int32),
        pltpu.VMEM((GRP, CHUNK), jnp.int32),
        pltpu.VMEM((GRP * CHUNK, HALF), jnp.float32),
        pltpu.VMEM((GRP * CHUNK,), jnp.int32),
        pltpu.VMEM((GRP, CHUNK), jnp.int32),
        pltpu.VMEM((GRP * CHUNK, HALF), jnp.float32),
        pltpu.SemaphoreType.DMA,
        pltpu.SemaphoreType.DMA,
        pltpu.SemaphoreType.DMA,
        pltpu.SemaphoreType.DMA,
    ],
)(_agg_body)


_deg_call = functools.partial(
    pl.kernel,
    out_type=jax.ShapeDtypeStruct((NSC, N_PAD, HALF), jnp.float32),
    mesh=_mesh,
    compiler_params=_sc_params,
    scratch_types=[
        pltpu.VMEM_SHARED((N_PAD, HALF), jnp.float32),
        pltpu.VMEM((GRP, CHUNK), jnp.int32),
        pltpu.VMEM((GRP, CHUNK), jnp.int32),
        pltpu.VMEM((CHUNK, HALF), jnp.float32),
        pltpu.SemaphoreType.DMA,
        pltpu.SemaphoreType.DMA,
    ],
)(_deg_body)


def _pre_body(x0_ref, x1_ref, dp_ref, w0_ref, w1_ref, bb_ref,
              hs_ref, dinv_ref):
    d = dp_ref[...]
    deg = d[0] + d[1] + 1.0
    dinvp = lax.rsqrt(deg)
    dinv_ref[...] = dinvp
    x0 = x0_ref[...]
    x1 = x1_ref[...]
    for c in range(NSC):
        h = jnp.maximum(x0 * w0_ref[c] + x1 * w1_ref[c] + bb_ref[c], 0.0)
        hs_ref[c] = dinvp * h


def _mid_body(m_ref, dinv_ref, b4_ref, bb_ref, hs_ref):
    dinvp = dinv_ref[...]
    dn = (((1,), (0,)), ((), ()))
    a0 = dinvp * m_ref[0]
    a1 = dinvp * m_ref[1]
    for c in range(NSC):
        q = (lax.dot_general(a0, b4_ref[c, 0], dn,
                             preferred_element_type=jnp.float32)
             + lax.dot_general(a1, b4_ref[c, 1], dn,
                               preferred_element_type=jnp.float32))
        h = jnp.maximum(q + bb_ref[c], 0.0)
        hs_ref[c] = dinvp * h


def _fin_body(m_ref, dinv_ref, b31_ref, bb31_ref, b32_ref, bb32_ref,
              wh_ref, sel_ref, bout_ref, out_ref):
    dinvp = dinv_ref[...]
    dn = (((1,), (0,)), ((), ()))
    a0 = dinvp * m_ref[0]
    a1 = dinvp * m_ref[1]
    acc = None
    for (b_ref, bb_ref, head) in ((b31_ref, bb31_ref, 0),
                                  (b32_ref, bb32_ref, 1)):
        for c in range(NSC):
            q = (lax.dot_general(a0, b_ref[c, 0], dn, preferred_element_type=jnp.float32)
                 + lax.dot_general(a1, b_ref[c, 1], dn,
                                   preferred_element_type=jnp.float32))
            h = jnp.maximum(q + bb_ref[c], 0.0)
            t = h * wh_ref[head, c]
            part = lax.dot_general(t, sel_ref[head], dn,
                                   preferred_element_type=jnp.float32)
            acc = part if acc is None else acc + part
    out_ref[...] = acc + bout_ref[...]


def _pk_spec():
    return pl.BlockSpec((PBLK, 128), lambda i: (i, 0))


def _pk2_spec():
    return pl.BlockSpec((NSC, PBLK, 128), lambda i: (0, i, 0))


def _full(shape):
    return pl.BlockSpec(shape, lambda i: (0,) * len(shape))


def _pre_call(x0, x1, dp, w0, w1, bb):
    return pl.pallas_call(
        _pre_body,
        grid=(NPK // PBLK,),
        in_specs=[_pk_spec(), _pk_spec(), _pk2_spec(),
                  _full((NSC, 1, 128)), _full((NSC, 1, 128)),
                  _full((NSC, 1, 128))],
        out_specs=[_pk2_spec(), _pk_spec()],
        out_shape=[
            jax.ShapeDtypeStruct((NSC, NPK, 128), jnp.float32),
            jax.ShapeDtypeStruct((NPK, 128), jnp.float32),
        ],
    )(x0, x1, dp, w0, w1, bb)


def _mid_call(mp, dinvp, b4, bb):
    return pl.pallas_call(
        _mid_body,
        grid=(NPK // PBLK,),
        in_specs=[_pk2_spec(), _pk_spec(),
                  _full((NSC, NSC, 128, 128)), _full((NSC, 1, 128))],
        out_specs=_pk2_spec(),
        out_shape=jax.ShapeDtypeStruct((NSC, NPK, 128), jnp.float32),
    )(mp, dinvp, b4, bb)


def _fin_call(mp, dinvp, b31, bb31, b32, bb32, wh, sel, bout):
    return pl.pallas_call(
        _fin_body,
        grid=(NPK // PBLK,),
        in_specs=[_pk2_spec(), _pk_spec(),
                  _full((NSC, NSC, 128, 128)), _full((NSC, 1, 128)),
                  _full((NSC, NSC, 128, 128)), _full((NSC, 1, 128)),
                  _full((2, NSC, 1, 128)), _full((2, 128, 16)),
                  _full((1, 16))],
        out_specs=pl.BlockSpec((PBLK, 16), lambda i: (i, 0)),
        out_shape=jax.ShapeDtypeStruct((NPK, 16), jnp.float32),
    )(mp, dinvp, b31, bb31, b32, bb32, wh, sel, bout)


def _blockdiag(w):
    eye = jnp.eye(8, dtype=jnp.float32)
    blocks = []
    for co in range(NSC):
        rows = []
        for ci in range(NSC):
            blk = w[co * HALF:(co + 1) * HALF, ci * HALF:(ci + 1) * HALF]
            rows.append(jnp.kron(eye, blk.T))
        blocks.append(jnp.stack(rows))
    return jnp.stack(blocks)


def _brc(v):
    t = v.reshape(NSC, 1, HALF)
    return jnp.tile(t, (1, 1, 8)).reshape(NSC, 1, 128)


_SEL = np.zeros((2, 128, 16), np.float32)
for _g in range(8):
    for _l in range(16):
        _SEL[0, _g * 16 + _l, 2 * _g] = 1.0
        _SEL[1, _g * 16 + _l, 2 * _g + 1] = 1.0


def _pack_nodes(v, fill):
    vp = jnp.full((N_PAD,), fill, v.dtype).at[:N].set(v)
    return jnp.repeat(vp, HALF).reshape(NPK, 128)


def kernel(x, edge_index, fc1_w, fc1_b, c1_w, c1_b, c2_w, c2_b,
           c31_w, c31_b, c32_w, c32_b, fc21_w, fc21_b, fc22_w, fc22_b):
    e = edge_index.shape[1]
    align = NSC * NT * CHUNK * GRP * 2
    e_pad = ((e + align - 1) // align) * align
    npad = e_pad - e
    ei = edge_index.astype(jnp.int32)
    pad_idx = (jnp.arange(npad, dtype=jnp.int32) % (N_PAD - N)) + N
    row1 = jnp.concatenate([ei[0], pad_idx])
    col2 = jnp.concatenate([ei[1], pad_idx]).reshape(e_pad // CHUNK, CHUNK)

    zeros = jnp.zeros((N_PAD, HALF), jnp.float32)
    ones = jnp.ones((CHUNK, HALF), jnp.float32)
    degp = _deg_call(col2, zeros, ones)
    dp = degp.reshape(NSC, NPK, 128)

    x0 = _pack_nodes(x[:, 0], 0.0)
    x1 = _pack_nodes(x[:, 1], 0.0)
    w0 = _brc(fc1_w[:, 0])
    w1 = _brc(fc1_w[:, 1])

    hsp, dinvp = _pre_call(x0, x1, dp, w0, w1, _brc(fc1_b))
    m = _agg_call(hsp.reshape(NSC, N_PAD, HALF), row1, col2)
    hsp = _mid_call(m.reshape(NSC, NPK, 128), dinvp, _blockdiag(c1_w),
                    _brc(c1_b))
    m = _agg_call(hsp.reshape(NSC, N_PAD, HALF), row1, col2)
    hsp = _mid_call(m.reshape(NSC, NPK, 128), dinvp, _blockdiag(c2_w),
                    _brc(c2_b))
    m = _agg_call(hsp.reshape(NSC, N_PAD, HALF), row1, col2)

    wh = jnp.stack([_brc(fc21_w[0]), _brc(fc22_w[0])])
    sel = jnp.asarray(_SEL)
    bout = jnp.tile(
        jnp.concatenate([fc21_b, fc22_b]).reshape(1, 2), (1, 8))
    outp = _fin_call(m.reshape(NSC, NPK, 128), dinvp, _blockdiag(c31_w),
                     _brc(c31_b), _blockdiag(c32_w), _brc(c32_b),
                     wh, sel, bout)
    return outp.reshape(N_PAD, 2)[:N]
